# Initial kernel scaffold; baseline (speedup 1.0000x reference)
import jax
import jax.numpy as jnp
from jax.experimental import pallas as pl


def kernel(edge_distance, source_element, target_element, W1, b1, src_emb, tgt_emb, W2, b2):
    raise NotImplementedError("write your pallas kernel here")


# speedup vs baseline: 2.0659x; 2.0659x over previous
import functools

import jax
import jax.numpy as jnp
from jax.experimental import pallas as pl
from jax.experimental.pallas import tpu as pltpu

IN_CHANNELS = 128
NUM_BASIS = 128
MAX_ELEM = 100
CUTOFF = 8.0
BLOCK_E = 1280


def _block_kernel(d_ref, src_ref, tgt_ref, w1_ref, b1_ref, tab_ref, w2_ref,
                  b2_ref, out_ref):
    b = d_ref.shape[0]
    step = CUTOFF / (IN_CHANNELS - 1)
    coeff = -0.5 / (step * step)
    offs = jax.lax.broadcasted_iota(
        jnp.int32, (b, IN_CHANNELS), 1).astype(jnp.float32) * step
    diff = d_ref[...] - offs
    gauss = jnp.exp(coeff * diff * diff)

    lane = jax.lax.broadcasted_iota(jnp.int32, (b, 2 * IN_CHANNELS), 1)
    src = src_ref[...]
    tgt = tgt_ref[...]
    oh = ((lane == src) | (lane == tgt + IN_CHANNELS)).astype(jnp.float32)

    acc = (jnp.dot(gauss, w1_ref[...], preferred_element_type=jnp.float32)
           + jnp.dot(oh, tab_ref[...], preferred_element_type=jnp.float32)
           + b1_ref[...])
    x = acc * jax.nn.sigmoid(acc)
    y = jnp.dot(x, w2_ref[...], preferred_element_type=jnp.float32) + b2_ref[...]
    out_ref[...] = y * jax.nn.sigmoid(y)


@jax.jit
def kernel(edge_distance, source_element, target_element, W1, b1, src_emb,
           tgt_emb, W2, b2):
    e = edge_distance.shape[0]
    nb = e // BLOCK_E
    d2 = edge_distance.reshape(e, 1)
    s2 = source_element.astype(jnp.int32).reshape(e, 1)
    t2 = target_element.astype(jnp.int32).reshape(e, 1)
    pad = ((0, IN_CHANNELS - MAX_ELEM), (0, 0))
    table = jnp.concatenate([jnp.pad(src_emb, pad), jnp.pad(tgt_emb, pad)], 0)

    grid = (nb,)
    row = lambda i: (i, 0)
    rep = lambda i: (0, 0)
    out = pl.pallas_call(
        _block_kernel,
        grid=grid,
        in_specs=[
            pl.BlockSpec((BLOCK_E, 1), row),
            pl.BlockSpec((BLOCK_E, 1), row),
            pl.BlockSpec((BLOCK_E, 1), row),
            pl.BlockSpec((IN_CHANNELS, NUM_BASIS), rep),
            pl.BlockSpec((1, NUM_BASIS), lambda i: (0, 0)),
            pl.BlockSpec((2 * IN_CHANNELS, NUM_BASIS), rep),
            pl.BlockSpec((NUM_BASIS, NUM_BASIS), rep),
            pl.BlockSpec((1, NUM_BASIS), lambda i: (0, 0)),
        ],
        out_specs=pl.BlockSpec((BLOCK_E, NUM_BASIS), row),
        out_shape=jax.ShapeDtypeStruct((e, NUM_BASIS), jnp.float32),
        compiler_params=pltpu.CompilerParams(
            dimension_semantics=("parallel",)),
    )(d2, s2, t2, W1, b1.reshape(1, -1), table, W2, b2.reshape(1, -1))
    return out
